# Initial kernel scaffold; baseline (speedup 1.0000x reference)
import functools

import jax
import jax.numpy as jnp
from jax import lax
from jax.experimental import pallas as pl
from jax.experimental.pallas import tpu as pltpu
from jax.experimental.pallas import tpu_sc as plsc

_TOK = 2048
_HID = 1024
_E = 8
_DFF = 2048
_CAP = 512

_NC = 2
_NS = 16
_LN = 16
_NW = _NC * _NS
_SLOTS = _E * _CAP
_SLOTS_PER = _SLOTS // _NW
_TOK_PER = _TOK // _NW
_SENT_SLOT = 2 * _SLOTS
_SENT_TOK = _TOK


def _cumsum0(a):
  n, m = a.shape
  s = 1
  while s < n:
    shifted = jnp.concatenate([jnp.zeros((s, m), a.dtype), a[: n - s]], axis=0)
    a = a + shifted
    s *= 2
  return a


def _gating_body(x_ref, wg_ref, idx_ref, w_ref, aux_ref, cnt_ref):
  x = x_ref[...]
  wg = wg_ref[...]
  logits = jnp.dot(x, wg, preferred_element_type=jnp.float32)
  m = jnp.max(logits, axis=-1, keepdims=True)
  ex = jnp.exp(logits - m)
  gates = ex / jnp.sum(ex, axis=-1, keepdims=True)

  lanes = lax.broadcasted_iota(jnp.int32, (_TOK, _E), 1)
  eq1 = logits == m
  idx1 = jnp.min(jnp.where(eq1, lanes, _E), axis=-1, keepdims=True)
  mask1 = lanes == idx1
  l2 = jnp.where(mask1, jnp.finfo(jnp.float32).min, logits)
  m2 = jnp.max(l2, axis=-1, keepdims=True)
  eq2 = l2 == m2
  idx2 = jnp.min(jnp.where(eq2, lanes, _E), axis=-1, keepdims=True)
  mask2 = lanes == idx2

  mask1f = mask1.astype(jnp.float32)
  mask2f = mask2.astype(jnp.float32)
  me = jnp.mean(gates, axis=0, keepdims=True)
  ce = jnp.mean(mask1f, axis=0, keepdims=True)
  aux_ref[0, 0] = jnp.sum(me * ce) / _E * (_E * _E)
  counts = jnp.sum(mask1f, axis=0, keepdims=True)
  cnt_ref[...] = counts.astype(jnp.int32)

  loc1 = _cumsum0(mask1f) - 1.0
  loc2 = _cumsum0(mask2f) - 1.0 + counts
  capf = jnp.float32(_CAP)
  mask1c = mask1f * (loc1 < capf)
  mask2c = mask2f * (loc2 < capf)
  pos1 = jnp.sum(loc1 * mask1c, axis=1, keepdims=True)
  pos2 = jnp.sum(loc2 * mask2c, axis=1, keepdims=True)
  keep1 = jnp.sum(mask1c, axis=1, keepdims=True)
  keep2 = jnp.sum(mask2c, axis=1, keepdims=True)
  gate1 = jnp.sum(gates * mask1c, axis=1, keepdims=True)
  gate2 = jnp.sum(gates * mask2c, axis=1, keepdims=True)
  denom = gate1 + gate2
  denom = jnp.where(denom > 1e-9, denom, 1e-9)
  wc1 = gate1 / denom * keep1
  wc2 = gate2 / denom * keep2

  posc1 = jnp.clip(pos1, 0.0, capf - 1.0).astype(jnp.int32)
  posc2 = jnp.clip(pos2, 0.0, capf - 1.0).astype(jnp.int32)
  flat1 = idx1 * _CAP + posc1
  flat2 = idx2 * _CAP + posc2
  s1 = jnp.where(keep1 > 0.0, flat1, _SENT_SLOT)
  s2 = jnp.where(keep2 > 0.0, flat2, _SENT_SLOT)

  idx_ref[...] = jnp.concatenate([s1, s2, flat1, flat2], axis=1)
  w_ref[...] = jnp.concatenate([wc1, wc2], axis=1)


def _gating(x, wg):
  return pl.pallas_call(
      _gating_body,
      out_shape=(
          jax.ShapeDtypeStruct((_TOK, 4), jnp.int32),
          jax.ShapeDtypeStruct((_TOK, 2), jnp.float32),
          jax.ShapeDtypeStruct((1, 1), jnp.float32),
          jax.ShapeDtypeStruct((1, _E), jnp.int32),
      ),
  )(x, wg)


def _ffn_body(xb_ref, w1_ref, w2_ref, y_ref):
  xb = xb_ref[...]
  h = jnp.maximum(jnp.dot(xb, w1_ref[0], preferred_element_type=jnp.float32), 0.0)
  y_ref[...] = jnp.dot(h, w2_ref[0], preferred_element_type=jnp.float32)


def _ffn(x_buf, w1, w2):
  return pl.pallas_call(
      _ffn_body,
      grid=(_E,),
      in_specs=[
          pl.BlockSpec((_CAP, _HID), lambda e: (e, 0)),
          pl.BlockSpec((1, _HID, _DFF), lambda e: (e, 0, 0)),
          pl.BlockSpec((1, _DFF, _HID), lambda e: (e, 0, 0)),
      ],
      out_specs=pl.BlockSpec((_CAP, _HID), lambda e: (e, 0)),
      out_shape=jax.ShapeDtypeStruct((_SLOTS, _HID), jnp.float32),
  )(x_buf, w1, w2)


_MESH = dict(core_axis_name="c", subcore_axis_name="s", num_cores=_NC,
             num_subcores=_NS)


def _dispatch_body(xpad_hbm, s1_hbm, s2_hbm, out_hbm, s1_v, s2_v, slot_tok,
                   rows_v, sem):
  wid = lax.axis_index("s") * _NC + lax.axis_index("c")
  base = wid * _SLOTS_PER
  pltpu.sync_copy(s1_hbm, s1_v)
  pltpu.sync_copy(s2_hbm, s2_v)
  for c in range(_SLOTS_PER // _LN):
    slot_tok[pl.ds(c * _LN, _LN)] = jnp.full((_LN,), _SENT_TOK, jnp.int32)

  def chunk(c, carry):
    tok = c * _LN + lax.iota(jnp.int32, _LN)
    for s_v in (s1_v, s2_v):
      v = s_v[pl.ds(c * _LN, _LN)]
      local = v - base
      msk = (v >= base) & (v < base + _SLOTS_PER)
      lc = jnp.clip(local, 0, _SLOTS_PER - 1)
      plsc.store_scatter(slot_tok, [lc], tok, mask=msk)
    return carry

  lax.fori_loop(0, _TOK // _LN, chunk, 0)

  half = _SLOTS_PER // 2
  for r in range(2):
    pltpu.async_copy(
        xpad_hbm.at[slot_tok.at[pl.ds(r * half, half)]], rows_v, sem).wait()
    pltpu.sync_copy(rows_v, out_hbm.at[pl.ds(base + r * half, half)])


def _dispatch(x_pad, s1, s2):
  half = _SLOTS_PER // 2
  kfn = functools.partial(
      pl.kernel,
      out_type=jax.ShapeDtypeStruct((_SLOTS, _HID), jnp.float32),
      mesh=plsc.VectorSubcoreMesh(**_MESH),
      scratch_types=[
          pltpu.VMEM((_TOK,), jnp.int32),
          pltpu.VMEM((_TOK,), jnp.int32),
          pltpu.VMEM((_SLOTS_PER,), jnp.int32),
          pltpu.VMEM((half, _HID), jnp.float32),
          pltpu.SemaphoreType.DMA,
      ],
  )(_dispatch_body)
  return kfn(x_pad, s1, s2)


def _combine_body(y_hbm, f1_hbm, f2_hbm, w1_hbm, w2_hbm, out_hbm, f1_v, f2_v,
                  w1_v, w2_v, y1_v, y2_v, o_v, sem):
  wid = lax.axis_index("s") * _NC + lax.axis_index("c")
  rows = _TOK_PER // 2

  for r in range(2):
    t0 = wid * _TOK_PER + r * rows
    pltpu.sync_copy(f1_hbm.at[pl.ds(t0, rows)], f1_v)
    pltpu.sync_copy(f2_hbm.at[pl.ds(t0, rows)], f2_v)
    pltpu.sync_copy(w1_hbm.at[pl.ds(t0, rows)], w1_v)
    pltpu.sync_copy(w2_hbm.at[pl.ds(t0, rows)], w2_v)
    pltpu.async_copy(y_hbm.at[f1_v], y1_v, sem).wait()
    pltpu.async_copy(y_hbm.at[f2_v], y2_v, sem).wait()

    def tok(i, carry):
      ii = jnp.full((_LN,), i, jnp.int32)
      wa = plsc.load_gather(w1_v, [ii])
      wb = plsc.load_gather(w2_v, [ii])

      def col(j, carry2):
        for u in range(4):
          cix = (j * 4 + u) * _LN + lax.iota(jnp.int32, _LN)
          a = plsc.load_gather(y1_v, [ii, cix])
          b = plsc.load_gather(y2_v, [ii, cix])
          plsc.store_scatter(o_v, [ii, cix], wa * a + wb * b)
        return carry2

      lax.fori_loop(0, _HID // _LN // 4, col, 0)
      return carry

    lax.fori_loop(0, rows, tok, 0)
    pltpu.sync_copy(o_v, out_hbm.at[pl.ds(t0, rows)])


def _combine(y, f1, f2, wc1, wc2):
  rows = _TOK_PER // 2
  kfn = functools.partial(
      pl.kernel,
      out_type=jax.ShapeDtypeStruct((_TOK, _HID), jnp.float32),
      mesh=plsc.VectorSubcoreMesh(**_MESH),
      scratch_types=[
          pltpu.VMEM((rows,), jnp.int32),
          pltpu.VMEM((rows,), jnp.int32),
          pltpu.VMEM((rows,), jnp.float32),
          pltpu.VMEM((rows,), jnp.float32),
          pltpu.VMEM((rows, _HID), jnp.float32),
          pltpu.VMEM((rows, _HID), jnp.float32),
          pltpu.VMEM((rows, _HID), jnp.float32),
          pltpu.SemaphoreType.DMA,
      ],
  )(_combine_body)
  return kfn(y, f1, f2, wc1, wc2)


@jax.jit
def kernel(hidden_states, Wg, W1, W2):
  idx_out, w_out, aux, cnt = _gating(hidden_states, Wg)
  s1 = idx_out[:, 0]
  s2 = idx_out[:, 1]
  f1 = idx_out[:, 2]
  f2 = idx_out[:, 3]
  wc1 = w_out[:, 0]
  wc2 = w_out[:, 1]

  x_pad = jnp.concatenate(
      [hidden_states, jnp.zeros((8, _HID), hidden_states.dtype)], axis=0)
  x_buf = _dispatch(x_pad, s1, s2)
  y = _ffn(x_buf, W1, W2)
  out = _combine(y, f1, f2, wc1, wc2)
  return out, aux[0, 0], cnt[0]


# speedup vs baseline: 1.0528x; 1.0528x over previous
import functools

import jax
import jax.numpy as jnp
from jax import lax
from jax.experimental import pallas as pl
from jax.experimental.pallas import tpu as pltpu
from jax.experimental.pallas import tpu_sc as plsc

_TOK = 2048
_HID = 1024
_E = 8
_DFF = 2048
_CAP = 512

_NC = 2
_NS = 16
_LN = 16
_NW = _NC * _NS
_SLOTS = _E * _CAP
_SLOTS_PER = _SLOTS // _NW
_TOK_PER = _TOK // _NW
_SENT_SLOT = 2 * _SLOTS
_SENT_TOK = _TOK


def _cumsum0(a):
  n, m = a.shape
  s = 1
  while s < n:
    shifted = jnp.concatenate([jnp.zeros((s, m), a.dtype), a[: n - s]], axis=0)
    a = a + shifted
    s *= 2
  return a


def _gating_body(x_ref, wg_ref, idx_ref, w_ref, aux_ref, cnt_ref):
  x = x_ref[...]
  wg = wg_ref[...]
  logits = jnp.dot(x, wg, preferred_element_type=jnp.float32)
  m = jnp.max(logits, axis=-1, keepdims=True)
  ex = jnp.exp(logits - m)
  gates = ex / jnp.sum(ex, axis=-1, keepdims=True)

  lanes = lax.broadcasted_iota(jnp.int32, (_TOK, _E), 1)
  eq1 = logits == m
  idx1 = jnp.min(jnp.where(eq1, lanes, _E), axis=-1, keepdims=True)
  mask1 = lanes == idx1
  l2 = jnp.where(mask1, jnp.finfo(jnp.float32).min, logits)
  m2 = jnp.max(l2, axis=-1, keepdims=True)
  eq2 = l2 == m2
  idx2 = jnp.min(jnp.where(eq2, lanes, _E), axis=-1, keepdims=True)
  mask2 = lanes == idx2

  mask1f = mask1.astype(jnp.float32)
  mask2f = mask2.astype(jnp.float32)
  me = jnp.mean(gates, axis=0, keepdims=True)
  ce = jnp.mean(mask1f, axis=0, keepdims=True)
  aux_ref[...] = jnp.sum(me * ce, axis=1, keepdims=True) / _E * (_E * _E)
  counts = jnp.sum(mask1f, axis=0, keepdims=True)
  cnt_ref[...] = counts.astype(jnp.int32)

  loc1 = _cumsum0(mask1f) - 1.0
  loc2 = _cumsum0(mask2f) - 1.0 + counts
  capf = jnp.float32(_CAP)
  mask1c = mask1f * (loc1 < capf)
  mask2c = mask2f * (loc2 < capf)
  pos1 = jnp.sum(loc1 * mask1c, axis=1, keepdims=True)
  pos2 = jnp.sum(loc2 * mask2c, axis=1, keepdims=True)
  keep1 = jnp.sum(mask1c, axis=1, keepdims=True)
  keep2 = jnp.sum(mask2c, axis=1, keepdims=True)
  gate1 = jnp.sum(gates * mask1c, axis=1, keepdims=True)
  gate2 = jnp.sum(gates * mask2c, axis=1, keepdims=True)
  denom = gate1 + gate2
  denom = jnp.where(denom > 1e-9, denom, 1e-9)
  wc1 = gate1 / denom * keep1
  wc2 = gate2 / denom * keep2

  posc1 = jnp.clip(pos1, 0.0, capf - 1.0).astype(jnp.int32)
  posc2 = jnp.clip(pos2, 0.0, capf - 1.0).astype(jnp.int32)
  flat1 = idx1 * _CAP + posc1
  flat2 = idx2 * _CAP + posc2
  s1 = jnp.where(keep1 > 0.0, flat1, _SENT_SLOT)
  s2 = jnp.where(keep2 > 0.0, flat2, _SENT_SLOT)

  idx_ref[...] = jnp.concatenate([s1, s2, flat1, flat2], axis=1)
  w_ref[...] = jnp.concatenate([wc1, wc2], axis=1)


def _gating(x, wg):
  return pl.pallas_call(
      _gating_body,
      out_shape=(
          jax.ShapeDtypeStruct((_TOK, 4), jnp.int32),
          jax.ShapeDtypeStruct((_TOK, 2), jnp.float32),
          jax.ShapeDtypeStruct((1, 1), jnp.float32),
          jax.ShapeDtypeStruct((1, _E), jnp.int32),
      ),
  )(x, wg)


def _ffn_body(xb_ref, w1_ref, w2_ref, y_ref):
  xb = xb_ref[...]
  h = jnp.maximum(jnp.dot(xb, w1_ref[0], preferred_element_type=jnp.float32), 0.0)
  y_ref[...] = jnp.dot(h, w2_ref[0], preferred_element_type=jnp.float32)


def _ffn(x_buf, w1, w2):
  return pl.pallas_call(
      _ffn_body,
      grid=(_E,),
      in_specs=[
          pl.BlockSpec((_CAP, _HID), lambda e: (e, 0)),
          pl.BlockSpec((1, _HID, _DFF), lambda e: (e, 0, 0)),
          pl.BlockSpec((1, _DFF, _HID), lambda e: (e, 0, 0)),
      ],
      out_specs=pl.BlockSpec((_CAP, _HID), lambda e: (e, 0)),
      out_shape=jax.ShapeDtypeStruct((_SLOTS, _HID), jnp.float32),
  )(x_buf, w1, w2)


_MESH = dict(core_axis_name="c", subcore_axis_name="s", num_cores=_NC,
             num_subcores=_NS)


def _dispatch_body(xpad_hbm, s1_hbm, s2_hbm, out_hbm, s1_v, s2_v, slot_tok,
                   rows_v, sem):
  wid = lax.axis_index("s") * _NC + lax.axis_index("c")
  base = wid * _SLOTS_PER
  pltpu.sync_copy(s1_hbm, s1_v)
  pltpu.sync_copy(s2_hbm, s2_v)
  for c in range(_SLOTS_PER // _LN):
    slot_tok[pl.ds(c * _LN, _LN)] = jnp.full((_LN,), _SENT_TOK, jnp.int32)

  def chunk(c, carry):
    tok = c * _LN + lax.iota(jnp.int32, _LN)
    for s_v in (s1_v, s2_v):
      v = s_v[pl.ds(c * _LN, _LN)]
      local = v - base
      msk = (v >= base) & (v < base + _SLOTS_PER)
      lc = jnp.clip(local, 0, _SLOTS_PER - 1)
      plsc.store_scatter(slot_tok, [lc], tok, mask=msk)
    return carry

  lax.fori_loop(0, _TOK // _LN, chunk, 0)

  half = _SLOTS_PER // 2
  for r in range(2):
    pltpu.async_copy(
        xpad_hbm.at[slot_tok.at[pl.ds(r * half, half)]], rows_v, sem).wait()
    pltpu.sync_copy(rows_v, out_hbm.at[pl.ds(base + r * half, half)])


def _dispatch(x_pad, s1, s2):
  half = _SLOTS_PER // 2
  kfn = functools.partial(
      pl.kernel,
      out_type=jax.ShapeDtypeStruct((_SLOTS, _HID), jnp.float32),
      mesh=plsc.VectorSubcoreMesh(**_MESH),
      scratch_types=[
          pltpu.VMEM((_TOK,), jnp.int32),
          pltpu.VMEM((_TOK,), jnp.int32),
          pltpu.VMEM((_SLOTS_PER,), jnp.int32),
          pltpu.VMEM((half, _HID), jnp.float32),
          pltpu.SemaphoreType.DMA,
      ],
      compiler_params=pltpu.CompilerParams(needs_layout_passes=False),
  )(_dispatch_body)
  return kfn(x_pad, s1, s2)


def _combine_body(y_hbm, f1_hbm, f2_hbm, w1_hbm, w2_hbm, out_hbm, f1_v, f2_v,
                  w1_v, w2_v, y1_v, y2_v, o_v, sem):
  wid = lax.axis_index("s") * _NC + lax.axis_index("c")
  rows = _TOK_PER // 2

  for r in range(2):
    t0 = wid * _TOK_PER + r * rows
    pltpu.sync_copy(f1_hbm.at[pl.ds(t0, rows)], f1_v)
    pltpu.sync_copy(f2_hbm.at[pl.ds(t0, rows)], f2_v)
    pltpu.sync_copy(w1_hbm.at[pl.ds(t0, rows)], w1_v)
    pltpu.sync_copy(w2_hbm.at[pl.ds(t0, rows)], w2_v)
    pltpu.async_copy(y_hbm.at[f1_v], y1_v, sem).wait()
    pltpu.async_copy(y_hbm.at[f2_v], y2_v, sem).wait()

    def tok(i, carry):
      ii = jnp.full((_LN,), i, jnp.int32)
      wa = plsc.load_gather(w1_v, [ii])
      wb = plsc.load_gather(w2_v, [ii])

      def col(j, carry2):
        for u in range(4):
          cix = (j * 4 + u) * _LN + lax.iota(jnp.int32, _LN)
          a = plsc.load_gather(y1_v, [ii, cix])
          b = plsc.load_gather(y2_v, [ii, cix])
          plsc.store_scatter(o_v, [ii, cix], wa * a + wb * b)
        return carry2

      lax.fori_loop(0, _HID // _LN // 4, col, 0)
      return carry

    lax.fori_loop(0, rows, tok, 0)
    pltpu.sync_copy(o_v, out_hbm.at[pl.ds(t0, rows)])


def _combine(y, f1, f2, wc1, wc2):
  rows = _TOK_PER // 2
  kfn = functools.partial(
      pl.kernel,
      out_type=jax.ShapeDtypeStruct((_TOK, _HID), jnp.float32),
      mesh=plsc.VectorSubcoreMesh(**_MESH),
      scratch_types=[
          pltpu.VMEM((rows,), jnp.int32),
          pltpu.VMEM((rows,), jnp.int32),
          pltpu.VMEM((rows,), jnp.float32),
          pltpu.VMEM((rows,), jnp.float32),
          pltpu.VMEM((rows, _HID), jnp.float32),
          pltpu.VMEM((rows, _HID), jnp.float32),
          pltpu.VMEM((rows, _HID), jnp.float32),
          pltpu.SemaphoreType.DMA,
      ],
      compiler_params=pltpu.CompilerParams(needs_layout_passes=False),
  )(_combine_body)
  return kfn(y, f1, f2, wc1, wc2)


@jax.jit
def kernel(hidden_states, Wg, W1, W2):
  idx_out, w_out, aux, cnt = _gating(hidden_states, Wg)
  s1 = idx_out[:, 0]
  s2 = idx_out[:, 1]
  f1 = idx_out[:, 2]
  f2 = idx_out[:, 3]
  wc1 = w_out[:, 0]
  wc2 = w_out[:, 1]

  x_pad = jnp.concatenate(
      [hidden_states, jnp.zeros((8, _HID), hidden_states.dtype)], axis=0)
  x_buf = _dispatch(x_pad, s1, s2)
  y = _ffn(x_buf, W1, W2)
  out = _combine(y, f1, f2, wc1, wc2)
  return out, aux[0, 0], cnt[0]
